# baseline (speedup 1.0000x reference)
import functools

import jax
import jax.numpy as jnp
from jax import lax
from jax.experimental import pallas as pl
from jax.experimental.pallas import tpu as pltpu
from jax.experimental.pallas import tpu_sc as plsc

NUM_CORES = 2
NUM_SUBCORES = 16
NW = NUM_CORES * NUM_SUBCORES
LANES = 16
B = 16384
D = 32
BPW = B // NW
IDX_CHUNK = 128
NCHUNK = BPW // IDX_CHUNK
ROWPACK = 128 // D
HALF = BPW // 2
NGROUP = HALF // LANES


def _mf_body(uidx_hbm, uoff_hbm, iidx_hbm, ioff_hbm, utab_hbm, itab_hbm,
             wb_hbm, out_hbm, uidx_v, iidx_v, uoff_v, ioff_v, urows_v,
             irows_v, wb_v, out_v, usem, isem):
    wid = lax.axis_index("s") * NUM_CORES + lax.axis_index("c")
    base = wid * BPW

    pltpu.sync_copy(uidx_hbm.at[wid], uidx_v)
    pltpu.sync_copy(iidx_hbm.at[wid], iidx_v)
    pltpu.sync_copy(uoff_hbm.at[pl.ds(base, BPW)], uoff_v)
    pltpu.sync_copy(ioff_hbm.at[pl.ds(base, BPW)], ioff_v)
    pltpu.sync_copy(wb_hbm, wb_v)

    w = wb_v[0, :]
    b = wb_v[1, :]
    lane = lax.iota(jnp.int32, 16)

    for half in range(2):
        copies = []
        for j in range(NCHUNK // 2):
            c = half * (NCHUNK // 2) + j
            dst = pl.ds(j * IDX_CHUNK, IDX_CHUNK)
            copies.append(pltpu.async_copy(utab_hbm.at[uidx_v.at[c]],
                                           urows_v.at[dst], usem))
            copies.append(pltpu.async_copy(itab_hbm.at[iidx_v.at[c]],
                                           irows_v.at[dst], isem))
        for cp in copies:
            cp.wait()

        def group(g, carry):
            row0 = pl.multiple_of(g * LANES, LANES)
            rows = row0 + lane
            uoff = uoff_v[pl.ds(half * HALF + row0, LANES)]
            ioff = ioff_v[pl.ds(half * HALF + row0, LANES)]
            acc = jnp.zeros((16,), jnp.float32)
            for d in range(D):
                uv = plsc.load_gather(urows_v, [rows, uoff + d])
                iv = plsc.load_gather(irows_v, [rows, ioff + d])
                acc = acc + uv * iv
            s = acc * w + b
            y = 1.0 / (1.0 + jnp.exp(-s))
            out_v[pl.ds(half * HALF + row0, LANES)] = y
            return carry

        lax.fori_loop(0, NGROUP, group, 0)

    pltpu.sync_copy(out_v, out_hbm.at[pl.ds(base, BPW)])


@jax.jit
def _mf_call(uidx, uoff, iidx, ioff, user_table4, item_table4, wb):
    mesh = plsc.VectorSubcoreMesh(core_axis_name="c", subcore_axis_name="s",
                                  num_cores=NUM_CORES,
                                  num_subcores=NUM_SUBCORES)
    fn = pl.kernel(
        _mf_body,
        out_type=jax.ShapeDtypeStruct((B,), jnp.float32),
        mesh=mesh,
        compiler_params=pltpu.CompilerParams(needs_layout_passes=False,
                                             use_tc_tiling_on_sc=True),
        scratch_types=[
            pltpu.VMEM((NCHUNK, IDX_CHUNK), jnp.int32),
            pltpu.VMEM((NCHUNK, IDX_CHUNK), jnp.int32),
            pltpu.VMEM((BPW,), jnp.int32),
            pltpu.VMEM((BPW,), jnp.int32),
            pltpu.VMEM((HALF, 128), jnp.float32),
            pltpu.VMEM((HALF, 128), jnp.float32),
            pltpu.VMEM((2, 16), jnp.float32),
            pltpu.VMEM((BPW,), jnp.float32),
            pltpu.SemaphoreType.DMA,
            pltpu.SemaphoreType.DMA,
        ],
    )
    return fn(uidx, uoff, iidx, ioff, user_table4, item_table4, wb)


def kernel(user_idx, item_idx, user_table, item_table, W_aff, b_aff):
    uidx_flat = user_idx.reshape(B).astype(jnp.int32)
    iidx_flat = item_idx.reshape(B).astype(jnp.int32)
    uidx = (uidx_flat // ROWPACK).reshape(NW, NCHUNK, IDX_CHUNK)
    iidx = (iidx_flat // ROWPACK).reshape(NW, NCHUNK, IDX_CHUNK)
    uoff = (uidx_flat % ROWPACK) * D
    ioff = (iidx_flat % ROWPACK) * D
    wb = jnp.stack([jnp.full((16,), W_aff[0, 0], jnp.float32),
                    jnp.full((16,), b_aff[0], jnp.float32)])
    ut4 = user_table.reshape(1000000 // ROWPACK, 128)
    it4 = item_table.reshape(1000000 // ROWPACK, 128)
    return _mf_call(uidx, uoff, iidx, ioff, ut4, it4, wb)
